# baseline (speedup 1.0000x reference)
import functools

import jax
import jax.numpy as jnp
import numpy as np
from jax import lax
from jax.experimental import pallas as pl
from jax.experimental.pallas import tpu as pltpu
from jax.experimental.pallas import tpu_sc as plsc

N = 10000
E = 320000
HID = 128
HEADS = 8
C = HID // HEADS
REL = 2
INV_W = 0.1
F32 = jnp.float32

NC = 2
NS = 16
NW = NC * NS
NPAD = 10112
DRP = 2 * NPAD
DR8 = DRP // 8
DR8P = 2560
CHK = 32
SUP = 128
NCH = SUP // CHK
EPT = 10112
EP = EPT * NW
NSUPER = EPT // SUP
TROWS = NPAD + DR8P
ROWS_T = TROWS // NS

PERM = np.array([(f % HEADS) * C + f // HEADS for f in range(HID)])

def _make_mesh():
    return plsc.VectorSubcoreMesh(core_axis_name="c", subcore_axis_name="s",
                                  num_cores=NC, num_subcores=NS)


def _sc_edge_body(src_h, dst_h, et_h, qt_h, kvt_h, qe_h,
                  ta_o,
                  srcb, dstb, etb, drb, scidx, qrows, kvrows, qerows, wc,
                  ta_sh, sem1, sem2, sem3):
    cid = lax.axis_index("c")
    sid = lax.axis_index("s")
    wid = cid * NS + sid

    zero16 = jnp.zeros((16,), F32)

    def zrow(i, _):
        for g in range(HID // 16):
            wc[i, pl.ds(g * 16, 16)] = zero16
        return 0

    lax.fori_loop(0, 2 * CHK, zrow, 0)

    za = sid * ROWS_T
    for t in range((ROWS_T + 2 * CHK - 1) // (2 * CHK)):
        nrows = min(2 * CHK, ROWS_T - t * 2 * CHK)
        pltpu.sync_copy(wc.at[pl.ds(0, nrows)],
                        ta_sh.at[pl.ds(za + t * 2 * CHK, nrows)])
    plsc.subcore_barrier()

    swp = lax.iota(jnp.int32, 16) ^ 8
    ebase = wid * EPT

    def superchunk(g, _):
        off = ebase + g * SUP
        pltpu.sync_copy(src_h.at[pl.ds(off, SUP)], srcb)
        pltpu.sync_copy(dst_h.at[pl.ds(off, SUP)], dstb)
        pltpu.sync_copy(et_h.at[pl.ds(off, SUP)], etb)
        for j in range(SUP // 16):
            dv = dstb[pl.ds(j * 16, 16)]
            rv = etb[pl.ds(j * 16, 16)]
            rv = jnp.minimum(jnp.maximum(rv, 0), REL - 1)
            dr = rv * NPAD + dv
            drb[pl.ds(j * 16, 16)] = dr
            h = j // 2
            half = j % 2
            scidx[h, pl.ds(half * 16, 16)] = dv
            scidx[h, pl.ds(CHK + half * 16, 16)] = (
                NPAD + lax.shift_right_logical(dr, 3))
        for h in range(NCH):
            cp1 = pltpu.async_copy(qt_h.at[dstb.at[pl.ds(h * CHK, CHK)]],
                                   qrows, sem1)
            cp2 = pltpu.async_copy(kvt_h.at[srcb.at[pl.ds(h * CHK, CHK)]],
                                   kvrows, sem2)
            cp3 = pltpu.async_copy(qe_h.at[drb.at[pl.ds(h * CHK, CHK)]],
                                   qerows, sem3)
            cp1.wait()
            cp2.wait()
            cp3.wait()

            def edge(j, _):
                t = qrows[j, pl.ds(0, 16)] * kvrows[j, pl.ds(0, 16)]
                for g2 in range(1, HID // 16):
                    t = t + qrows[j, pl.ds(g2 * 16, 16)] * kvrows[j, pl.ds(g2 * 16, 16)]
                u = t + lax.gather(
                    t, swp[:, None],
                    lax.GatherDimensionNumbers(offset_dims=(), collapsed_slice_dims=(0,),
                                               start_index_map=(0,)),
                    (1,), mode=lax.GatherScatterMode.PROMISE_IN_BOUNDS)
                ex = jnp.exp(u + qerows[j, pl.ds(0, 16)])
                slotf = qerows[j, pl.ds(16, 16)]
                for g2 in range(HID // 16):
                    wc[j, pl.ds(g2 * 16, 16)] = (
                        kvrows[j, pl.ds(HID + g2 * 16, 16)] * ex)
                    wc[CHK + j, pl.ds(g2 * 16, 16)] = jnp.where(
                        slotf == float(g2), ex, zero16)
                return 0

            lax.fori_loop(0, CHK, edge, 0)
            pltpu.sync_copy(wc, ta_sh.at[scidx.at[h]], add=True)
        return 0

    lax.fori_loop(0, NSUPER, superchunk, 0)
    plsc.subcore_barrier()

    oa = cid * TROWS + sid * ROWS_T
    for t in range((ROWS_T + 2 * CHK - 1) // (2 * CHK)):
        nrows = min(2 * CHK, ROWS_T - t * 2 * CHK)
        pltpu.sync_copy(ta_sh.at[pl.ds(sid * ROWS_T + t * 2 * CHK, nrows)],
                        ta_o.at[pl.ds(oa + t * 2 * CHK, nrows)])


_SC_EDGE_CACHE = []


def _sc_edge(*args):
    if not _SC_EDGE_CACHE:
        _SC_EDGE_CACHE.append(_build_sc_edge())
    return _SC_EDGE_CACHE[0](*args)


def _build_sc_edge():
    return functools.partial(
        pl.kernel,
        out_type=jax.ShapeDtypeStruct((NC * TROWS, HID), F32),
        mesh=_make_mesh(),
        scratch_types=[
            pltpu.VMEM((SUP,), jnp.int32),
            pltpu.VMEM((SUP,), jnp.int32),
            pltpu.VMEM((SUP,), jnp.int32),
            pltpu.VMEM((SUP,), jnp.int32),
            pltpu.VMEM((NCH, 2 * CHK), jnp.int32),
            pltpu.VMEM((CHK, HID), F32),
            pltpu.VMEM((CHK, 2 * HID), F32),
            pltpu.VMEM((CHK, HID), F32),
            pltpu.VMEM((2 * CHK, HID), F32),
            pltpu.VMEM_SHARED((TROWS, HID), F32),
            pltpu.SemaphoreType.DMA,
            pltpu.SemaphoreType.DMA,
            pltpu.SemaphoreType.DMA,
        ],
    )(_sc_edge_body)


B = 1000
GRID = N // B

LRELU = 0.01


def _lrelu(x):
    return jnp.where(x > 0, x, x * LRELU)


def _head_mask64():
    r = lax.broadcasted_iota(jnp.int32, (64, 8), 0)
    c = lax.broadcasted_iota(jnp.int32, (64, 8), 1)
    return (r // 8 == c).astype(F32)


def _head_mask128():
    r = lax.broadcasted_iota(jnp.int32, (HID, 8), 0)
    c = lax.broadcasted_iota(jnp.int32, (HID, 8), 1)
    return (r % 8 == c).astype(F32)


def _bcast16():
    r = lax.broadcasted_iota(jnp.int32, (16, HID), 0)
    c = lax.broadcasted_iota(jnp.int32, (16, HID), 1)
    return (r % 8 == c % 8).astype(F32) * 0.5


def _tc1_body(desc_ref, tw_ref, np_ref, cp_ref,
              wd, bd, wt, bt, wn, bn, wc, bc,
              winv, binv, wspec, bspec,
              wqm, bqm, wkm, bkm, wvm, bvm, wom, bom, lng, lnb,
              c2hw, c2hb,
              wq1, bq1, wk1, bk1, wv1, bv1, ws1, bs1, ek1,
              q1_o, k1_o, v1_o, s1_o, qe0_o, qe1_o, aux_o):
    mods = [
        _lrelu(jnp.dot(desc_ref[...], wd[...], preferred_element_type=F32) + bd[...]),
        _lrelu(jnp.dot(tw_ref[...], wt[...], preferred_element_type=F32) + bt[...]),
        _lrelu(jnp.dot(np_ref[...], wn[...], preferred_element_type=F32) + bn[...]),
        _lrelu(jnp.dot(cp_ref[...], wc[...], preferred_element_type=F32) + bc[...]),
    ]
    invs, specs, toks = [], [], []
    for i in range(4):
        inv = jnp.tanh(jnp.dot(mods[i], winv[i], preferred_element_type=F32) + binv[i])
        spec = _lrelu(jnp.dot(mods[i], wspec[i], preferred_element_type=F32) + bspec[i])
        invs.append(inv)
        specs.append(spec)
        toks.append(jnp.concatenate((inv, spec), axis=1))
    mh = _head_mask64()
    qs = [jnp.dot(t, wqm[...], preferred_element_type=F32) + bqm[...] for t in toks]
    ks = [jnp.dot(t, wkm[...], preferred_element_type=F32) + bkm[...] for t in toks]
    vs = [jnp.dot(t, wvm[...], preferred_element_type=F32) + bvm[...] for t in toks]
    scale = 1.0 / np.sqrt(8.0)
    ct_out = []
    for l in range(4):
        s_lm = [jnp.dot(qs[l] * ks[m], mh, preferred_element_type=F32) * scale
                for m in range(4)]
        mx = jnp.maximum(jnp.maximum(s_lm[0], s_lm[1]),
                         jnp.maximum(s_lm[2], s_lm[3]))
        e_lm = [jnp.exp(s - mx) for s in s_lm]
        ssum = e_lm[0] + e_lm[1] + e_lm[2] + e_lm[3]
        o_l = 0.0
        for m in range(4):
            a = e_lm[m] / ssum
            o_l = o_l + jnp.dot(a, mh.T, preferred_element_type=F32) * vs[m]
        att = jnp.dot(o_l, wom[...], preferred_element_type=F32) + bom[...]
        x = att + toks[l]
        mu = jnp.mean(x, axis=1, keepdims=True)
        var = jnp.mean((x - mu) ** 2, axis=1, keepdims=True)
        ct_out.append((x - mu) / jnp.sqrt(var + 1e-5) * lng[...] + lnb[...])
    cmean = (ct_out[0] + ct_out[1] + ct_out[2] + ct_out[3]) * 0.25
    fused = _lrelu(jnp.dot(cmean, c2hw[...], preferred_element_type=F32) + c2hb[...])

    q1 = jnp.dot(fused, wq1[...], preferred_element_type=F32) + bq1[...]
    k1 = jnp.dot(fused, wk1[...], preferred_element_type=F32) + bk1[...]
    v1 = jnp.dot(fused, wv1[...], preferred_element_type=F32) + bv1[...]
    s1 = jnp.dot(fused, ws1[...], preferred_element_type=F32) + bs1[...]
    q1_o[...] = q1
    k1_o[...] = k1
    v1_o[...] = v1
    s1_o[...] = s1
    mh128 = _head_mask128()
    nb = q1.shape[0]
    zpad = jnp.zeros((nb, HID - 32), F32)
    d_id = pl.program_id(0) * B + lax.broadcasted_iota(jnp.int32, (nb, 16), 0)
    for r, ref in ((0, qe0_o), (1, qe1_o)):
        qe8 = jnp.dot(q1 * ek1[r], mh128, preferred_element_type=F32)
        slotf = ((r * NPAD + d_id) % 8).astype(F32)
        ref[...] = jnp.concatenate((qe8, qe8, slotf, zpad), axis=1)

    center = (invs[0] + invs[1] + invs[2] + invs[3]) * 0.25
    inv_ss = 0.0
    for i in range(4):
        d = invs[i] - center
        inv_ss = inv_ss + jnp.sum(d * d)
    nrm = [jnp.maximum(jnp.sqrt(jnp.sum(s * s, axis=1, keepdims=True)), 1e-8)
           for s in specs]
    ov_ss = 0.0
    for l in range(4):
        for r in range(l + 1, 4):
            dot = jnp.sum(specs[l] * specs[r], axis=1, keepdims=True)
            ov_ss = ov_ss + jnp.sum(jnp.abs(dot / (nrm[l] * nrm[r])))
    lane = lax.broadcasted_iota(jnp.int32, (1, HID), 1)
    vec = jnp.where(lane == 0, inv_ss, jnp.where(lane == 1, ov_ss, 0.0))

    @pl.when(pl.program_id(0) == 0)
    def _():
        aux_o[...] = vec

    @pl.when(pl.program_id(0) != 0)
    def _():
        aux_o[...] = aux_o[...] + vec


def _combine(accs, saccs, ev, skip):
    bc = _bcast16()
    acc = accs[0] + accs[1]
    s0 = saccs[0, 0] + saccs[1, 0]
    s1 = saccs[0, 1] + saccs[1, 1]
    b0 = jnp.dot(s0, bc, preferred_element_type=F32)
    b1 = jnp.dot(s1, bc, preferred_element_type=F32)
    term = b0 * ev[0] + b1 * ev[1]
    denom = b0 + b1 + 1e-16
    return (acc + term) / denom + skip


def _tc2_body(accs_ref, saccs_ref, skip1_ref,
              ev1, wq2, bq2, wk2, bk2, wv2, bv2, ws2, bs2, ek2,
              q2_o, k2_o, v2_o, s2_o, qe0_o, qe1_o):
    out = _combine(accs_ref[...], saccs_ref[...], ev1[...], skip1_ref[...])
    x1 = _lrelu(out)
    q2 = jnp.dot(x1, wq2[...], preferred_element_type=F32) + bq2[...]
    k2 = jnp.dot(x1, wk2[...], preferred_element_type=F32) + bk2[...]
    v2 = jnp.dot(x1, wv2[...], preferred_element_type=F32) + bv2[...]
    s2 = jnp.dot(x1, ws2[...], preferred_element_type=F32) + bs2[...]
    q2_o[...] = q2
    k2_o[...] = k2
    v2_o[...] = v2
    s2_o[...] = s2
    mh128 = _head_mask128()
    nb = q2.shape[0]
    zpad = jnp.zeros((nb, HID - 32), F32)
    d_id = pl.program_id(0) * B + lax.broadcasted_iota(jnp.int32, (nb, 16), 0)
    for r, ref in ((0, qe0_o), (1, qe1_o)):
        qe8 = jnp.dot(q2 * ek2[r], mh128, preferred_element_type=F32)
        slotf = ((r * NPAD + d_id) % 8).astype(F32)
        ref[...] = jnp.concatenate((qe8, qe8, slotf, zpad), axis=1)


def _tc3_body(accs_ref, saccs_ref, skip2_ref,
              ev2, womlp, bomlp, whead, bhead,
              lp_o):
    out = _combine(accs_ref[...], saccs_ref[...], ev2[...], skip2_ref[...])
    y = _lrelu(jnp.dot(out, womlp[...], preferred_element_type=F32) + bomlp[...])
    lp_o[...] = jnp.dot(y, whead[...], preferred_element_type=F32) + bhead[...]


def _full(shape):
    nd = len(shape)
    return pl.BlockSpec(shape, lambda i: (0,) * nd)


def _rows(shape):
    nd = len(shape)
    return pl.BlockSpec(shape, lambda i, _nd=nd: (i,) + (0,) * (_nd - 1))


def _tc1_call(desc, tw, np8, cp8, w):
    out_shapes = (
        jax.ShapeDtypeStruct((NPAD, HID), F32),
        jax.ShapeDtypeStruct((NPAD, HID), F32),
        jax.ShapeDtypeStruct((NPAD, HID), F32),
        jax.ShapeDtypeStruct((NPAD, HID), F32),
        jax.ShapeDtypeStruct((NPAD, HID), F32),
        jax.ShapeDtypeStruct((NPAD, HID), F32),
        jax.ShapeDtypeStruct((1, HID), F32),
    )
    in_specs = [
        _rows((B, 768)), _rows((B, 768)), _rows((B, 8)), _rows((B, 8)),
        _full((768, 32)), _full((32,)), _full((768, 32)), _full((32,)),
        _full((8, 32)), _full((32,)), _full((8, 32)), _full((32,)),
        _full((4, 32, 32)), _full((4, 32)), _full((4, 32, 32)), _full((4, 32)),
        _full((64, 64)), _full((64,)), _full((64, 64)), _full((64,)),
        _full((64, 64)), _full((64,)), _full((64, 64)), _full((64,)),
        _full((64,)), _full((64,)),
        _full((64, HID)), _full((HID,)),
        _full((HID, HID)), _full((HID,)), _full((HID, HID)), _full((HID,)),
        _full((HID, HID)), _full((HID,)), _full((HID, HID)), _full((HID,)),
        _full((REL, HID)),
    ]
    out_specs = (
        _rows((B, HID)), _rows((B, HID)), _rows((B, HID)), _rows((B, HID)),
        _rows((B, HID)), _rows((B, HID)), _full((1, HID)),
    )
    return pl.pallas_call(
        _tc1_body, grid=(GRID,),
        in_specs=in_specs, out_specs=out_specs, out_shape=out_shapes,
    )(desc, tw, np8, cp8, *w)


def _tc2_call(accs, saccs, skip1, w):
    out_shapes = (
        jax.ShapeDtypeStruct((NPAD, HID), F32),
        jax.ShapeDtypeStruct((NPAD, HID), F32),
        jax.ShapeDtypeStruct((NPAD, HID), F32),
        jax.ShapeDtypeStruct((NPAD, HID), F32),
        jax.ShapeDtypeStruct((NPAD, HID), F32),
        jax.ShapeDtypeStruct((NPAD, HID), F32),
    )
    in_specs = [
        pl.BlockSpec((NC, B, HID), lambda i: (0, i, 0)),
        pl.BlockSpec((NC, REL, B, 16), lambda i: (0, 0, i, 0)),
        _rows((B, HID)),
        _full((REL, HID)),
        _full((HID, HID)), _full((HID,)), _full((HID, HID)), _full((HID,)),
        _full((HID, HID)), _full((HID,)), _full((HID, HID)), _full((HID,)),
        _full((REL, HID)),
    ]
    out_specs = (
        _rows((B, HID)), _rows((B, HID)), _rows((B, HID)), _rows((B, HID)),
        _rows((B, HID)), _rows((B, HID)),
    )
    return pl.pallas_call(
        _tc2_body, grid=(GRID,),
        in_specs=in_specs, out_specs=out_specs, out_shape=out_shapes,
    )(accs, saccs, skip1, *w)


def _tc3_call(accs, saccs, skip2, w):
    out_shapes = jax.ShapeDtypeStruct((NPAD, HID), F32)
    in_specs = [
        pl.BlockSpec((NC, B, HID), lambda i: (0, i, 0)),
        pl.BlockSpec((NC, REL, B, 16), lambda i: (0, 0, i, 0)),
        _rows((B, HID)),
        _full((REL, HID)),
        _full((HID, HID)), _full((HID,)), _full((HID, HID)), _full((HID,)),
    ]
    return pl.pallas_call(
        _tc3_body, grid=(GRID,),
        in_specs=in_specs, out_specs=(_rows((B, HID)),), out_shape=(out_shapes,),
    )(accs, saccs, skip2, *w)[0]


def _conv_weights(p, rel_emb, first):
    perm = PERM
    wq, wk, wv, ws = p['wq'], p['wk'], p['wv'], p['wskip']
    bq, bk, bv, bs = p['bq'], p['bk'], p['bv'], p['bskip']
    if not first:
        wq, wk, wv, ws = (w[perm, :] for w in (wq, wk, wv, ws))
    sc = 1.0 / np.sqrt(C)
    ek = (rel_emb @ p['we'])[:, perm]
    return [wq[:, perm] * sc, bq[perm] * sc,
            wk[:, perm], bk[perm],
            wv[:, perm], bv[perm],
            ws[:, perm], bs[perm],
            ek], ek


def kernel(description, tweet, num_prop, cat_prop, edge_index, edge_type, params):
    np8 = jnp.pad(num_prop, ((0, 0), (0, 3)))
    cp8 = jnp.pad(cat_prop, ((0, 0), (0, 5)))

    src = edge_index[0].astype(jnp.int32)
    dst = edge_index[1].astype(jnp.int32)
    et = edge_type.astype(jnp.int32)
    padn = EP - E
    padidx = N + (jnp.arange(padn, dtype=jnp.int32) % 16)
    srcp = jnp.concatenate([src, padidx])
    dstp = jnp.concatenate([dst, padidx])
    etp = jnp.concatenate([et, jnp.zeros((padn,), jnp.int32)])

    pm = params['mha']
    w1, ek1 = _conv_weights(params['conv1'], params['rel_emb'], True)
    w2, ek2 = _conv_weights(params['conv2'], params['rel_emb'], False)

    tc1_w = [
        params['desc']['w'], params['desc']['b'],
        params['tweet']['w'], params['tweet']['b'],
        jnp.pad(params['num']['w'], ((0, 3), (0, 0))), params['num']['b'],
        jnp.pad(params['cat']['w'], ((0, 5), (0, 0))), params['cat']['b'],
        jnp.stack([p['w'] for p in params['inv']]),
        jnp.stack([p['b'] for p in params['inv']]),
        jnp.stack([p['w'] for p in params['spec']]),
        jnp.stack([p['b'] for p in params['spec']]),
        pm['wq'], pm['bq'], pm['wk'], pm['bk'], pm['wv'], pm['bv'],
        pm['wo'], pm['bo'], params['ln_g'], params['ln_b'],
        params['c2h']['w'], params['c2h']['b'],
    ] + w1

    q1, k1, v1, s1, qe0, qe1, auxp = _tc1_call(description, tweet, np8, cp8, tc1_w)
    qe_t1 = jnp.concatenate([qe0, qe1], axis=0)

    kv1 = jnp.concatenate([k1, v1], axis=1)
    ta1 = _sc_edge(srcp, dstp, etp, q1, kv1, qe_t1).reshape(NC, TROWS, HID)
    accs1 = ta1[:, :NPAD]
    saccs1 = ta1[:, NPAD:NPAD + DR8].reshape(NC, REL, NPAD, 16)

    q2, k2, v2, s2, qe0b, qe1b = _tc2_call(accs1, saccs1, s1, [ek1] + w2)
    qe_t2 = jnp.concatenate([qe0b, qe1b], axis=0)

    kv2 = jnp.concatenate([k2, v2], axis=1)
    ta2 = _sc_edge(srcp, dstp, etp, q2, kv2, qe_t2).reshape(NC, TROWS, HID)
    accs2 = ta2[:, :NPAD]
    saccs2 = ta2[:, NPAD:NPAD + DR8].reshape(NC, REL, NPAD, 16)

    whp = jnp.pad(params['head']['w'], ((0, 0), (0, HID - 2)))
    bhp = jnp.pad(params['head']['b'], ((0, HID - 2),))
    tc3_w = [ek2, params['outmlp']['w'][PERM, :], params['outmlp']['b'], whp, bhp]
    lp = _tc3_call(accs2, saccs2, s2, tc3_w)

    logits = lp[:N, :2]
    inv_ss = auxp[0, 0]
    ov_ss = auxp[0, 1]
    aux = INV_W * (inv_ss / (N * 4 * 32) + 0.5 * ov_ss / (N * 6))
    return logits, aux


# speedup vs baseline: 1.3512x; 1.3512x over previous
import functools

import jax
import jax.numpy as jnp
import numpy as np
from jax import lax
from jax.experimental import pallas as pl
from jax.experimental.pallas import tpu as pltpu
from jax.experimental.pallas import tpu_sc as plsc

N = 10000
E = 320000
HID = 128
HEADS = 8
C = HID // HEADS
REL = 2
INV_W = 0.1
F32 = jnp.float32

NC = 2
NS = 16
NW = NC * NS
NPAD = 10112
DRP = 2 * NPAD
DR16 = DRP // 16
DR16P = 1280
CHK = 16
SUP = 128
NCH = SUP // CHK
EPT = 10112
EP = EPT * NW
NSUPER = EPT // SUP
TROWS = NPAD + DR16P
ROWS_T = TROWS // NS

PERM = np.array([(f % HEADS) * C + f // HEADS for f in range(HID)])

def _make_mesh():
    return plsc.VectorSubcoreMesh(core_axis_name="c", subcore_axis_name="s",
                                  num_cores=NC, num_subcores=NS)


def _sc_edge_body(src_h, dst_h, et_h, qt_h, kvt_h, qe_h,
                  ta_o,
                  srcb, dstb, etb, drb, scidx,
                  qrows0, qrows1, kvrows0, kvrows1, qerows0, qerows1,
                  wc0, wc1, ta_sh,
                  semq0, semq1, semkv0, semkv1, semqe0, semqe1, scs0, scs1):
    cid = lax.axis_index("c")
    sid = lax.axis_index("s")
    wid = cid * NS + sid

    qrows = (qrows0, qrows1)
    kvrows = (kvrows0, kvrows1)
    qerows = (qerows0, qerows1)
    wc = (wc0, wc1)
    semq = (semq0, semq1)
    semkv = (semkv0, semkv1)
    semqe = (semqe0, semqe1)
    scs = (scs0, scs1)

    zero16 = jnp.zeros((16,), F32)

    def zrow(i, _):
        for g in range(HID // 16):
            wc0[i, pl.ds(g * 16, 16)] = zero16
        return 0

    lax.fori_loop(0, 2 * CHK, zrow, 0)

    za = sid * ROWS_T
    for t in range((ROWS_T + 2 * CHK - 1) // (2 * CHK)):
        nrows = min(2 * CHK, ROWS_T - t * 2 * CHK)
        pltpu.sync_copy(wc0.at[pl.ds(0, nrows)],
                        ta_sh.at[pl.ds(za + t * 2 * CHK, nrows)])
    plsc.subcore_barrier()

    swp = lax.iota(jnp.int32, 16) ^ 8
    ebase = wid * EPT

    def _issue(h, b):
        cps = (
            pltpu.async_copy(qt_h.at[dstb.at[pl.ds(h * CHK, CHK)]],
                             qrows[b], semq[b]),
            pltpu.async_copy(kvt_h.at[srcb.at[pl.ds(h * CHK, CHK)]],
                             kvrows[b], semkv[b]),
            pltpu.async_copy(qe_h.at[drb.at[pl.ds(h * CHK, CHK)]],
                             qerows[b], semqe[b]),
        )
        return cps

    def _wait_scatter(b):
        pltpu.make_async_copy(wc[b], ta_sh.at[scidx.at[b]], scs[b]).wait()

    def superchunk(g, _):
        @pl.when(g > 0)
        def _():
            _wait_scatter(0)
            _wait_scatter(1)

        off = ebase + g * SUP
        pltpu.sync_copy(src_h.at[pl.ds(off, SUP)], srcb)
        pltpu.sync_copy(dst_h.at[pl.ds(off, SUP)], dstb)
        pltpu.sync_copy(et_h.at[pl.ds(off, SUP)], etb)
        for j in range(SUP // 16):
            dv = dstb[pl.ds(j * 16, 16)]
            rv = etb[pl.ds(j * 16, 16)]
            rv = jnp.minimum(jnp.maximum(rv, 0), REL - 1)
            dr = rv * NPAD + dv
            drb[pl.ds(j * 16, 16)] = dr
            scidx[j, pl.ds(0, 16)] = dv
            scidx[j, pl.ds(16, 16)] = NPAD + lax.shift_right_logical(dr, 4)

        gath = _issue(0, 0)
        for h in range(NCH):
            b = h % 2
            if h + 1 < NCH:
                nxt = _issue(h + 1, 1 - b)
            for cp in gath:
                cp.wait()
            if h >= 2:
                _wait_scatter(b)

            qr, kvr, qer, w = qrows[b], kvrows[b], qerows[b], wc[b]
            hi8 = jnp.where(lax.iota(jnp.int32, 16) >= 8, 1.0, 0.0).astype(F32)

            def edge(j, _):
                t = qr[j, pl.ds(0, 16)] * kvr[j, pl.ds(0, 16)]
                for g2 in range(1, HID // 16):
                    t = t + qr[j, pl.ds(g2 * 16, 16)] * kvr[j, pl.ds(g2 * 16, 16)]
                u = t + lax.gather(
                    t, swp[:, None],
                    lax.GatherDimensionNumbers(offset_dims=(), collapsed_slice_dims=(0,),
                                               start_index_map=(0,)),
                    (1,), mode=lax.GatherScatterMode.PROMISE_IN_BOUNDS)
                ex = jnp.exp(u + qer[j, pl.ds(0, 16)])
                slotf = qer[j, pl.ds(16, 16)]
                for g2 in range(HID // 16):
                    w[j, pl.ds(g2 * 16, 16)] = (
                        kvr[j, pl.ds(HID + g2 * 16, 16)] * ex)
                    w[CHK + j, pl.ds(g2 * 16, 16)] = jnp.where(
                        slotf == hi8 + float(2 * g2), ex, zero16)
                return 0

            lax.fori_loop(0, CHK, edge, 0)
            pltpu.async_copy(w, ta_sh.at[scidx.at[h]], scs[b], add=True)
            if h + 1 < NCH:
                gath = nxt
        return 0

    lax.fori_loop(0, NSUPER, superchunk, 0)
    _wait_scatter(0)
    _wait_scatter(1)
    plsc.subcore_barrier()

    oa = cid * TROWS + sid * ROWS_T
    for t in range((ROWS_T + 2 * CHK - 1) // (2 * CHK)):
        nrows = min(2 * CHK, ROWS_T - t * 2 * CHK)
        pltpu.sync_copy(ta_sh.at[pl.ds(sid * ROWS_T + t * 2 * CHK, nrows)],
                        ta_o.at[pl.ds(oa + t * 2 * CHK, nrows)])


_SC_EDGE_CACHE = []


def _sc_edge(*args):
    if not _SC_EDGE_CACHE:
        _SC_EDGE_CACHE.append(_build_sc_edge())
    return _SC_EDGE_CACHE[0](*args)


def _build_sc_edge():
    return functools.partial(
        pl.kernel,
        out_type=jax.ShapeDtypeStruct((NC * TROWS, HID), F32),
        mesh=_make_mesh(),
        scratch_types=[
            pltpu.VMEM((SUP,), jnp.int32),
            pltpu.VMEM((SUP,), jnp.int32),
            pltpu.VMEM((SUP,), jnp.int32),
            pltpu.VMEM((SUP,), jnp.int32),
            pltpu.VMEM((NCH, 2 * CHK), jnp.int32),
            pltpu.VMEM((CHK, HID), F32),
            pltpu.VMEM((CHK, HID), F32),
            pltpu.VMEM((CHK, 2 * HID), F32),
            pltpu.VMEM((CHK, 2 * HID), F32),
            pltpu.VMEM((CHK, HID), F32),
            pltpu.VMEM((CHK, HID), F32),
            pltpu.VMEM((2 * CHK, HID), F32),
            pltpu.VMEM((2 * CHK, HID), F32),
            pltpu.VMEM_SHARED((TROWS, HID), F32),
            pltpu.SemaphoreType.DMA,
            pltpu.SemaphoreType.DMA,
            pltpu.SemaphoreType.DMA,
            pltpu.SemaphoreType.DMA,
            pltpu.SemaphoreType.DMA,
            pltpu.SemaphoreType.DMA,
            pltpu.SemaphoreType.DMA,
            pltpu.SemaphoreType.DMA,
        ],
    )(_sc_edge_body)


B = 1000
GRID = N // B

LRELU = 0.01


def _lrelu(x):
    return jnp.where(x > 0, x, x * LRELU)


def _head_mask64():
    r = lax.broadcasted_iota(jnp.int32, (64, 8), 0)
    c = lax.broadcasted_iota(jnp.int32, (64, 8), 1)
    return (r // 8 == c).astype(F32)


def _head_mask128():
    r = lax.broadcasted_iota(jnp.int32, (HID, 8), 0)
    c = lax.broadcasted_iota(jnp.int32, (HID, 8), 1)
    return (r % 8 == c).astype(F32)


def _bcast8():
    r = lax.broadcasted_iota(jnp.int32, (8, HID), 0)
    c = lax.broadcasted_iota(jnp.int32, (8, HID), 1)
    return (r == c % 8).astype(F32)


def _tc1_body(desc_ref, tw_ref, np_ref, cp_ref,
              wd, bd, wt, bt, wn, bn, wc, bc,
              winv, binv, wspec, bspec,
              wqm, bqm, wkm, bkm, wvm, bvm, wom, bom, lng, lnb,
              c2hw, c2hb,
              wq1, bq1, wk1, bk1, wv1, bv1, ws1, bs1, ek1,
              q1_o, k1_o, v1_o, s1_o, qe0_o, qe1_o, aux_o):
    mods = [
        _lrelu(jnp.dot(desc_ref[...], wd[...], preferred_element_type=F32) + bd[...]),
        _lrelu(jnp.dot(tw_ref[...], wt[...], preferred_element_type=F32) + bt[...]),
        _lrelu(jnp.dot(np_ref[...], wn[...], preferred_element_type=F32) + bn[...]),
        _lrelu(jnp.dot(cp_ref[...], wc[...], preferred_element_type=F32) + bc[...]),
    ]
    invs, specs, toks = [], [], []
    for i in range(4):
        inv = jnp.tanh(jnp.dot(mods[i], winv[i], preferred_element_type=F32) + binv[i])
        spec = _lrelu(jnp.dot(mods[i], wspec[i], preferred_element_type=F32) + bspec[i])
        invs.append(inv)
        specs.append(spec)
        toks.append(jnp.concatenate((inv, spec), axis=1))
    mh = _head_mask64()
    qs = [jnp.dot(t, wqm[...], preferred_element_type=F32) + bqm[...] for t in toks]
    ks = [jnp.dot(t, wkm[...], preferred_element_type=F32) + bkm[...] for t in toks]
    vs = [jnp.dot(t, wvm[...], preferred_element_type=F32) + bvm[...] for t in toks]
    scale = 1.0 / np.sqrt(8.0)
    ct_out = []
    for l in range(4):
        s_lm = [jnp.dot(qs[l] * ks[m], mh, preferred_element_type=F32) * scale
                for m in range(4)]
        mx = jnp.maximum(jnp.maximum(s_lm[0], s_lm[1]),
                         jnp.maximum(s_lm[2], s_lm[3]))
        e_lm = [jnp.exp(s - mx) for s in s_lm]
        ssum = e_lm[0] + e_lm[1] + e_lm[2] + e_lm[3]
        o_l = 0.0
        for m in range(4):
            a = e_lm[m] / ssum
            o_l = o_l + jnp.dot(a, mh.T, preferred_element_type=F32) * vs[m]
        att = jnp.dot(o_l, wom[...], preferred_element_type=F32) + bom[...]
        x = att + toks[l]
        mu = jnp.mean(x, axis=1, keepdims=True)
        var = jnp.mean((x - mu) ** 2, axis=1, keepdims=True)
        ct_out.append((x - mu) / jnp.sqrt(var + 1e-5) * lng[...] + lnb[...])
    cmean = (ct_out[0] + ct_out[1] + ct_out[2] + ct_out[3]) * 0.25
    fused = _lrelu(jnp.dot(cmean, c2hw[...], preferred_element_type=F32) + c2hb[...])

    q1 = jnp.dot(fused, wq1[...], preferred_element_type=F32) + bq1[...]
    k1 = jnp.dot(fused, wk1[...], preferred_element_type=F32) + bk1[...]
    v1 = jnp.dot(fused, wv1[...], preferred_element_type=F32) + bv1[...]
    s1 = jnp.dot(fused, ws1[...], preferred_element_type=F32) + bs1[...]
    q1_o[...] = q1
    k1_o[...] = k1
    v1_o[...] = v1
    s1_o[...] = s1
    mh128 = _head_mask128()
    nb = q1.shape[0]
    zpad = jnp.zeros((nb, HID - 32), F32)
    d_id = pl.program_id(0) * B + lax.broadcasted_iota(jnp.int32, (nb, 16), 0)
    for r, ref in ((0, qe0_o), (1, qe1_o)):
        qe8 = jnp.dot(q1 * ek1[r], mh128, preferred_element_type=F32)
        slotf = ((r * NPAD + d_id) % 16).astype(F32)
        ref[...] = jnp.concatenate((qe8, qe8, slotf, zpad), axis=1)

    center = (invs[0] + invs[1] + invs[2] + invs[3]) * 0.25
    inv_ss = 0.0
    for i in range(4):
        d = invs[i] - center
        inv_ss = inv_ss + jnp.sum(d * d)
    nrm = [jnp.maximum(jnp.sqrt(jnp.sum(s * s, axis=1, keepdims=True)), 1e-8)
           for s in specs]
    ov_ss = 0.0
    for l in range(4):
        for r in range(l + 1, 4):
            dot = jnp.sum(specs[l] * specs[r], axis=1, keepdims=True)
            ov_ss = ov_ss + jnp.sum(jnp.abs(dot / (nrm[l] * nrm[r])))
    lane = lax.broadcasted_iota(jnp.int32, (1, HID), 1)
    vec = jnp.where(lane == 0, inv_ss, jnp.where(lane == 1, ov_ss, 0.0))

    @pl.when(pl.program_id(0) == 0)
    def _():
        aux_o[...] = vec

    @pl.when(pl.program_id(0) != 0)
    def _():
        aux_o[...] = aux_o[...] + vec


def _combine(accs, saccs, ev, skip):
    bc = _bcast8()
    acc = accs[0] + accs[1]
    s0 = saccs[0, 0] + saccs[1, 0]
    s1 = saccs[0, 1] + saccs[1, 1]
    b0 = jnp.dot(s0, bc, preferred_element_type=F32)
    b1 = jnp.dot(s1, bc, preferred_element_type=F32)
    term = b0 * ev[0] + b1 * ev[1]
    denom = b0 + b1 + 1e-16
    return (acc + term) / denom + skip


def _tc2_body(accs_ref, saccs_ref, skip1_ref,
              ev1, wq2, bq2, wk2, bk2, wv2, bv2, ws2, bs2, ek2,
              q2_o, k2_o, v2_o, s2_o, qe0_o, qe1_o):
    out = _combine(accs_ref[...], saccs_ref[...], ev1[...], skip1_ref[...])
    x1 = _lrelu(out)
    q2 = jnp.dot(x1, wq2[...], preferred_element_type=F32) + bq2[...]
    k2 = jnp.dot(x1, wk2[...], preferred_element_type=F32) + bk2[...]
    v2 = jnp.dot(x1, wv2[...], preferred_element_type=F32) + bv2[...]
    s2 = jnp.dot(x1, ws2[...], preferred_element_type=F32) + bs2[...]
    q2_o[...] = q2
    k2_o[...] = k2
    v2_o[...] = v2
    s2_o[...] = s2
    mh128 = _head_mask128()
    nb = q2.shape[0]
    zpad = jnp.zeros((nb, HID - 32), F32)
    d_id = pl.program_id(0) * B + lax.broadcasted_iota(jnp.int32, (nb, 16), 0)
    for r, ref in ((0, qe0_o), (1, qe1_o)):
        qe8 = jnp.dot(q2 * ek2[r], mh128, preferred_element_type=F32)
        slotf = ((r * NPAD + d_id) % 16).astype(F32)
        ref[...] = jnp.concatenate((qe8, qe8, slotf, zpad), axis=1)


def _tc3_body(accs_ref, saccs_ref, skip2_ref,
              ev2, womlp, bomlp, whead, bhead,
              lp_o):
    out = _combine(accs_ref[...], saccs_ref[...], ev2[...], skip2_ref[...])
    y = _lrelu(jnp.dot(out, womlp[...], preferred_element_type=F32) + bomlp[...])
    lp_o[...] = jnp.dot(y, whead[...], preferred_element_type=F32) + bhead[...]


def _full(shape):
    nd = len(shape)
    return pl.BlockSpec(shape, lambda i: (0,) * nd)


def _rows(shape):
    nd = len(shape)
    return pl.BlockSpec(shape, lambda i, _nd=nd: (i,) + (0,) * (_nd - 1))


def _tc1_call(desc, tw, np8, cp8, w):
    out_shapes = (
        jax.ShapeDtypeStruct((NPAD, HID), F32),
        jax.ShapeDtypeStruct((NPAD, HID), F32),
        jax.ShapeDtypeStruct((NPAD, HID), F32),
        jax.ShapeDtypeStruct((NPAD, HID), F32),
        jax.ShapeDtypeStruct((NPAD, HID), F32),
        jax.ShapeDtypeStruct((NPAD, HID), F32),
        jax.ShapeDtypeStruct((1, HID), F32),
    )
    in_specs = [
        _rows((B, 768)), _rows((B, 768)), _rows((B, 8)), _rows((B, 8)),
        _full((768, 32)), _full((32,)), _full((768, 32)), _full((32,)),
        _full((8, 32)), _full((32,)), _full((8, 32)), _full((32,)),
        _full((4, 32, 32)), _full((4, 32)), _full((4, 32, 32)), _full((4, 32)),
        _full((64, 64)), _full((64,)), _full((64, 64)), _full((64,)),
        _full((64, 64)), _full((64,)), _full((64, 64)), _full((64,)),
        _full((64,)), _full((64,)),
        _full((64, HID)), _full((HID,)),
        _full((HID, HID)), _full((HID,)), _full((HID, HID)), _full((HID,)),
        _full((HID, HID)), _full((HID,)), _full((HID, HID)), _full((HID,)),
        _full((REL, HID)),
    ]
    out_specs = (
        _rows((B, HID)), _rows((B, HID)), _rows((B, HID)), _rows((B, HID)),
        _rows((B, HID)), _rows((B, HID)), _full((1, HID)),
    )
    return pl.pallas_call(
        _tc1_body, grid=(GRID,),
        in_specs=in_specs, out_specs=out_specs, out_shape=out_shapes,
    )(desc, tw, np8, cp8, *w)


def _tc2_call(accs, saccs, skip1, w):
    out_shapes = (
        jax.ShapeDtypeStruct((NPAD, HID), F32),
        jax.ShapeDtypeStruct((NPAD, HID), F32),
        jax.ShapeDtypeStruct((NPAD, HID), F32),
        jax.ShapeDtypeStruct((NPAD, HID), F32),
        jax.ShapeDtypeStruct((NPAD, HID), F32),
        jax.ShapeDtypeStruct((NPAD, HID), F32),
    )
    in_specs = [
        pl.BlockSpec((NC, B, HID), lambda i: (0, i, 0)),
        pl.BlockSpec((NC, REL, B, 8), lambda i: (0, 0, i, 0)),
        _rows((B, HID)),
        _full((REL, HID)),
        _full((HID, HID)), _full((HID,)), _full((HID, HID)), _full((HID,)),
        _full((HID, HID)), _full((HID,)), _full((HID, HID)), _full((HID,)),
        _full((REL, HID)),
    ]
    out_specs = (
        _rows((B, HID)), _rows((B, HID)), _rows((B, HID)), _rows((B, HID)),
        _rows((B, HID)), _rows((B, HID)),
    )
    return pl.pallas_call(
        _tc2_body, grid=(GRID,),
        in_specs=in_specs, out_specs=out_specs, out_shape=out_shapes,
    )(accs, saccs, skip1, *w)


def _tc3_call(accs, saccs, skip2, w):
    out_shapes = jax.ShapeDtypeStruct((NPAD, HID), F32)
    in_specs = [
        pl.BlockSpec((NC, B, HID), lambda i: (0, i, 0)),
        pl.BlockSpec((NC, REL, B, 8), lambda i: (0, 0, i, 0)),
        _rows((B, HID)),
        _full((REL, HID)),
        _full((HID, HID)), _full((HID,)), _full((HID, HID)), _full((HID,)),
    ]
    return pl.pallas_call(
        _tc3_body, grid=(GRID,),
        in_specs=in_specs, out_specs=(_rows((B, HID)),), out_shape=(out_shapes,),
    )(accs, saccs, skip2, *w)[0]


def _conv_weights(p, rel_emb, first):
    perm = PERM
    wq, wk, wv, ws = p['wq'], p['wk'], p['wv'], p['wskip']
    bq, bk, bv, bs = p['bq'], p['bk'], p['bv'], p['bskip']
    if not first:
        wq, wk, wv, ws = (w[perm, :] for w in (wq, wk, wv, ws))
    sc = 1.0 / np.sqrt(C)
    ek = (rel_emb @ p['we'])[:, perm]
    return [wq[:, perm] * sc, bq[perm] * sc,
            wk[:, perm], bk[perm],
            wv[:, perm], bv[perm],
            ws[:, perm], bs[perm],
            ek], ek


def kernel(description, tweet, num_prop, cat_prop, edge_index, edge_type, params):
    np8 = jnp.pad(num_prop, ((0, 0), (0, 3)))
    cp8 = jnp.pad(cat_prop, ((0, 0), (0, 5)))

    src = edge_index[0].astype(jnp.int32)
    dst = edge_index[1].astype(jnp.int32)
    et = edge_type.astype(jnp.int32)
    padn = EP - E
    padidx = N + (jnp.arange(padn, dtype=jnp.int32) % 16)
    srcp = jnp.concatenate([src, padidx])
    dstp = jnp.concatenate([dst, padidx])
    etp = jnp.concatenate([et, jnp.zeros((padn,), jnp.int32)])

    pm = params['mha']
    w1, ek1 = _conv_weights(params['conv1'], params['rel_emb'], True)
    w2, ek2 = _conv_weights(params['conv2'], params['rel_emb'], False)

    tc1_w = [
        params['desc']['w'], params['desc']['b'],
        params['tweet']['w'], params['tweet']['b'],
        jnp.pad(params['num']['w'], ((0, 3), (0, 0))), params['num']['b'],
        jnp.pad(params['cat']['w'], ((0, 5), (0, 0))), params['cat']['b'],
        jnp.stack([p['w'] for p in params['inv']]),
        jnp.stack([p['b'] for p in params['inv']]),
        jnp.stack([p['w'] for p in params['spec']]),
        jnp.stack([p['b'] for p in params['spec']]),
        pm['wq'], pm['bq'], pm['wk'], pm['bk'], pm['wv'], pm['bv'],
        pm['wo'], pm['bo'], params['ln_g'], params['ln_b'],
        params['c2h']['w'], params['c2h']['b'],
    ] + w1

    q1, k1, v1, s1, qe0, qe1, auxp = _tc1_call(description, tweet, np8, cp8, tc1_w)
    qe_t1 = jnp.concatenate([qe0, qe1], axis=0)

    kv1 = jnp.concatenate([k1, v1], axis=1)
    ta1 = _sc_edge(srcp, dstp, etp, q1, kv1, qe_t1).reshape(NC, TROWS, HID)
    accs1 = ta1[:, :NPAD]
    saccs1 = ta1[:, NPAD:NPAD + DR16].reshape(NC, REL, NPAD, 8)

    q2, k2, v2, s2, qe0b, qe1b = _tc2_call(accs1, saccs1, s1, [ek1] + w2)
    qe_t2 = jnp.concatenate([qe0b, qe1b], axis=0)

    kv2 = jnp.concatenate([k2, v2], axis=1)
    ta2 = _sc_edge(srcp, dstp, etp, q2, kv2, qe_t2).reshape(NC, TROWS, HID)
    accs2 = ta2[:, :NPAD]
    saccs2 = ta2[:, NPAD:NPAD + DR16].reshape(NC, REL, NPAD, 8)

    whp = jnp.pad(params['head']['w'], ((0, 0), (0, HID - 2)))
    bhp = jnp.pad(params['head']['b'], ((0, HID - 2),))
    tc3_w = [ek2, params['outmlp']['w'][PERM, :], params['outmlp']['b'], whp, bhp]
    lp = _tc3_call(accs2, saccs2, s2, tc3_w)

    logits = lp[:N, :2]
    inv_ss = auxp[0, 0]
    ov_ss = auxp[0, 1]
    aux = INV_W * (inv_ss / (N * 4 * 32) + 0.5 * ov_ss / (N * 6))
    return logits, aux
